# speedup vs baseline: 1.0143x; 1.0143x over previous
import functools

import jax
import jax.numpy as jnp
import numpy as np
from jax.experimental import pallas as pl
from jax.experimental.pallas import tpu as pltpu

INTERPRET = False


def _pick_be(E):
    for be in (2560, 2048, 1280, 1024, 512, 256, 128, 64, 32, 16, 8):
        if E % be == 0:
            return be
    return E


def _edge_kernel_body(gd_ref, gs_ref, he_ref, rel_ref,
                      C_ref, wd2_ref, We2_ref, be2_ref,
                      Wc1_ref, bc1_ref, Wc2_ref, bc2_ref,
                      Weu_ref, beu_ref,
                      m_ref, relcw_ref, heo_ref):
    gd = gd_ref[...]
    gs = gs_ref[...]
    he = he_ref[...]
    rel = rel_ref[...]
    d2 = jnp.sum(rel * rel, axis=-1, keepdims=True)
    t1 = gd + gs + jnp.dot(he, C_ref[...], preferred_element_type=jnp.float32)
    t1 = t1 + d2 * wd2_ref[...]
    a = jax.nn.silu(t1)
    m = jnp.dot(a, We2_ref[...], preferred_element_type=jnp.float32) + be2_ref[...]
    u = jax.nn.silu(jnp.dot(m, Wc1_ref[...], preferred_element_type=jnp.float32) + bc1_ref[...])
    cw = jnp.tanh(jnp.dot(u, Wc2_ref[...], preferred_element_type=jnp.float32) + bc2_ref[...])
    m_ref[...] = m
    relcw_ref[...] = rel * cw
    heo_ref[...] = he + jnp.dot(m, Weu_ref[...], preferred_element_type=jnp.float32) + beu_ref[...]


def _edge_kernel_body_last(gd_ref, gs_ref, he_ref, rel_ref,
                           C_ref, wd2_ref, We2_ref, be2_ref,
                           Wc1_ref, bc1_ref, Wc2_ref, bc2_ref,
                           Weu_ref, beu_ref, Wed1_ref, bed1_ref, Wed2_ref, bed2_ref,
                           m_ref, relcw_ref, pe_ref):
    gd = gd_ref[...]
    gs = gs_ref[...]
    he = he_ref[...]
    rel = rel_ref[...]
    d2 = jnp.sum(rel * rel, axis=-1, keepdims=True)
    t1 = gd + gs + jnp.dot(he, C_ref[...], preferred_element_type=jnp.float32)
    t1 = t1 + d2 * wd2_ref[...]
    a = jax.nn.silu(t1)
    m = jnp.dot(a, We2_ref[...], preferred_element_type=jnp.float32) + be2_ref[...]
    u = jax.nn.silu(jnp.dot(m, Wc1_ref[...], preferred_element_type=jnp.float32) + bc1_ref[...])
    cw = jnp.tanh(jnp.dot(u, Wc2_ref[...], preferred_element_type=jnp.float32) + bc2_ref[...])
    m_ref[...] = m
    relcw_ref[...] = rel * cw
    heo = he + jnp.dot(m, Weu_ref[...], preferred_element_type=jnp.float32) + beu_ref[...]
    pe = jnp.maximum(jnp.dot(heo, Wed1_ref[...], preferred_element_type=jnp.float32) + bed1_ref[...], 0.0)
    pe_ref[...] = jnp.dot(pe, Wed2_ref[...], preferred_element_type=jnp.float32) + bed2_ref[...]


def _edge_layer(gd, gs, he, rel, C, wd2, We2, be2, Wc1, bc1, Wc2, bc2,
                Weu, beu, last, Wed1=None, bed1=None, Wed2=None, bed2=None):
    E = gd.shape[0]
    ND = gd.shape[1]
    ED = he.shape[1]
    BE = _pick_be(E)
    nb = E // BE
    row = lambda b: (b, 0)
    full = lambda b: (0, 0)
    in_specs = [
        pl.BlockSpec((BE, ND), row),
        pl.BlockSpec((BE, ND), row),
        pl.BlockSpec((BE, ED), row),
        pl.BlockSpec((BE, 3), row),
        pl.BlockSpec((ED, ND), full),
        pl.BlockSpec((1, ND), full),
        pl.BlockSpec((ND, ND), full),
        pl.BlockSpec((1, ND), full),
        pl.BlockSpec((ND, ND), full),
        pl.BlockSpec((1, ND), full),
        pl.BlockSpec((ND, 1), full),
        pl.BlockSpec((1, 1), full),
        pl.BlockSpec((ND, ED), full),
        pl.BlockSpec((1, ED), full),
    ]
    args = [gd, gs, he, rel, C, wd2, We2, be2, Wc1, bc1, Wc2, bc2, Weu, beu]
    out_shapes = [
        jax.ShapeDtypeStruct((E, ND), jnp.float32),
        jax.ShapeDtypeStruct((E, 3), jnp.float32),
    ]
    out_specs = [
        pl.BlockSpec((BE, ND), row),
        pl.BlockSpec((BE, 3), row),
    ]
    if last:
        ET = Wed2.shape[1]
        in_specs += [
            pl.BlockSpec((ED, ED), full),
            pl.BlockSpec((1, ED), full),
            pl.BlockSpec((ED, ET), full),
            pl.BlockSpec((1, ET), full),
        ]
        args += [Wed1, bed1, Wed2, bed2]
        out_shapes.append(jax.ShapeDtypeStruct((E, ET), jnp.float32))
        out_specs.append(pl.BlockSpec((BE, ET), row))
        body = _edge_kernel_body_last
    else:
        out_shapes.append(jax.ShapeDtypeStruct((E, ED), jnp.float32))
        out_specs.append(pl.BlockSpec((BE, ED), row))
        body = _edge_kernel_body
    return pl.pallas_call(
        body,
        grid=(nb,),
        in_specs=in_specs,
        out_specs=out_specs,
        out_shape=out_shapes,
        interpret=INTERPRET,
    )(*args)


def _smear(tt, T, TD):
    offsets = jnp.linspace(0.0, float(T), TD)
    coeff = -0.5 / (offsets[1] - offsets[0]) ** 2
    return jnp.exp(coeff * (tt.astype(jnp.float32)[:, None] - offsets[None, :]) ** 2)


def _ssp(x):
    return jax.nn.softplus(x) - np.log(2.0)


def kernel(protein_pos, protein_v, h_node_pert, pos_pert, h_edge_pert, Wp, bp, Wn, We, We1, be1, We2, be2, Wc1, bc1, Wc2, bc2, Wh1, bh1, Wh2, bh2, Weu, beu, Wv1, bv1, Wv2, bv2, Wnd1, bnd1, Wnd2, bnd2, Wed1, bed1, Wed2, bed2, batch_protein, batch_node, batch_edge, edge_index, t):
    NP = protein_pos.shape[0]
    NL = pos_pert.shape[0]
    E = h_edge_pert.shape[0]
    B = t.shape[0]
    ND = Wp.shape[1]
    TD = ND - Wn.shape[1]
    L = We1.shape[0]
    T = 1000

    src = edge_index[0]
    dst = edge_index[1]

    cnt = jnp.clip(jax.ops.segment_sum(jnp.ones((NP, 1)), batch_protein, num_segments=B), 1.0)
    offset = jax.ops.segment_sum(protein_pos, batch_protein, num_segments=B) / cnt
    x = pos_pert - offset[batch_node]
    smear_t = _smear(t, T, TD)
    h = jnp.concatenate([h_node_pert @ Wn, smear_t[batch_node]], axis=-1)
    he = jnp.concatenate([h_edge_pert @ We, smear_t[batch_edge]], axis=-1)
    hp = protein_v @ Wp + bp
    ctx = jax.ops.segment_sum(hp, batch_protein, num_segments=B) / cnt
    h = h + ctx[batch_node]
    deg = jnp.clip(jax.ops.segment_sum(jnp.ones((E, 1)), dst, num_segments=NL), 1.0)

    pred_edge = None
    for l in range(L):
        hA = h @ We1[l, :ND] + be1[l]
        hB = h @ We1[l, ND:2 * ND]
        gd = hA[dst]
        gs = hB[src]
        rel = x[dst] - x[src]
        C = We1[l, 2 * ND:-1]
        wd2 = We1[l, -1:]
        last = l == L - 1
        outs = _edge_layer(gd, gs, he, rel, C, wd2, We2[l], be2[l][None, :],
                           Wc1[l], bc1[l][None, :], Wc2[l], bc2[l][None, :],
                           Weu[l], beu[l][None, :], last,
                           Wed1, bed1[None, :], Wed2, bed2[None, :])
        m, relcw = outs[0], outs[1]
        x = x + jax.ops.segment_sum(relcw, dst, num_segments=NL) / deg
        agg = jax.ops.segment_sum(m, dst, num_segments=NL)
        h = h + jax.nn.silu(jnp.concatenate([h, agg], axis=-1) @ Wh1[l] + bh1[l]) @ Wh2[l] + bh2[l]
        if last:
            pred_edge = outs[2]
        else:
            he = outs[2]

    v = _ssp(h @ Wv1 + bv1) @ Wv2 + bv2
    pred_node = jax.nn.relu(v @ Wnd1 + bnd1) @ Wnd2 + bnd2
    pred_pos = x + offset[batch_node]
    return (pred_node, pred_pos, pred_edge)


# speedup vs baseline: 1.2077x; 1.1907x over previous
import functools

import jax
import jax.numpy as jnp
import numpy as np
from jax import lax
from jax.experimental import pallas as pl
from jax.experimental.pallas import tpu as pltpu
from jax.experimental.pallas import tpu_sc as plsc

INTERPRET = False

_NTILES = 16
_CH = 2


def _pick_be(E):
    for be in (2560, 2048, 1280, 1024, 512, 256, 128, 64, 32, 16, 8):
        if E % be == 0:
            return be
    return E


def _sc_scatter(m, aux, dstp, NL):
    E = m.shape[0]
    NCH = E // (_CH * 128)
    base, extra = NCH // _NTILES, NCH % _NTILES
    n8 = NL // 8
    b8, e8 = n8 // _NTILES, n8 % _NTILES
    mesh = plsc.VectorSubcoreMesh(core_axis_name="c", subcore_axis_name="s")

    @functools.partial(
        pl.kernel,
        out_type=(jax.ShapeDtypeStruct((NL, 128), jnp.float32),
                  jax.ShapeDtypeStruct((NL, 128), jnp.float32)),
        mesh=mesh,
        scratch_types=[
            pltpu.VMEM((_CH * 128, 128), jnp.float32),
            pltpu.VMEM((_CH, 128), jnp.int32),
            pltpu.VMEM_SHARED((NL, 128), jnp.float32),
        ],
    )
    def scat(m_hbm, aux_hbm, dstp_hbm, out_m, out_aux, pay_v, idx_v, accum):
        c = lax.axis_index("c")
        s = lax.axis_index("s")

        def zrow(i, _):
            for j in range(8):
                pay_v[i, pl.ds(j * 16, 16)] = jnp.zeros((16,), jnp.float32)
            return 0
        lax.fori_loop(0, _CH * 128, zrow, 0)
        row0 = pl.multiple_of(8 * (b8 * s + jnp.minimum(s, e8)), 8)
        nrow_lo, nrow_hi = 8 * b8, 8 * (b8 + 1)

        def zfill(nrow):
            pr = _CH * 128
            nfull = (nrow // pr) * pr
            for off in range(0, nfull, pr):
                pltpu.sync_copy(pay_v, accum.at[pl.ds(row0 + off, pr)])
            if nrow - nfull:
                pltpu.sync_copy(pay_v.at[pl.ds(0, nrow - nfull)],
                                accum.at[pl.ds(row0 + nfull, nrow - nfull)])

        @pl.when(s < e8)
        def _():
            zfill(nrow_hi)

        @pl.when(s >= e8)
        def _():
            zfill(nrow_lo)

        plsc.subcore_barrier()

        start = base * s + jnp.minimum(s, extra)
        stop = start + base + jnp.where(s < extra, 1, 0)

        def make_body(pay_hbm):
            def body(ch, _):
                pltpu.sync_copy(dstp_hbm.at[ch], idx_v)
                off = pl.multiple_of(ch * (_CH * 128), 8)
                pltpu.sync_copy(pay_hbm.at[pl.ds(off, _CH * 128)], pay_v)
                for j in range(_CH):
                    pltpu.sync_copy(pay_v.at[pl.ds(j * 128, 128)],
                                    accum.at[idx_v.at[j]], add=True)
                return 0
            return body

        @pl.when(c == 0)
        def _():
            lax.fori_loop(start, stop, make_body(m_hbm), 0)

        @pl.when(c == 1)
        def _():
            lax.fori_loop(start, stop, make_body(aux_hbm), 0)

        plsc.subcore_barrier()

        def wout(out, nrow):
            pltpu.sync_copy(accum.at[pl.ds(row0, nrow)], out.at[pl.ds(row0, nrow)])

        @pl.when((c == 0) & (s < e8))
        def _():
            wout(out_m, nrow_hi)

        @pl.when((c == 0) & (s >= e8))
        def _():
            wout(out_m, nrow_lo)

        @pl.when((c == 1) & (s < e8))
        def _():
            wout(out_aux, nrow_hi)

        @pl.when((c == 1) & (s >= e8))
        def _():
            wout(out_aux, nrow_lo)

    return scat(m, aux, dstp)


def _edge_kernel_body(gd_ref, gs_ref, he_ref, rel_ref,
                      C_ref, wd2_ref, We2_ref, be2_ref,
                      Wc1_ref, bc1_ref, Wc2_ref, bc2_ref,
                      Weu_ref, beu_ref,
                      m_ref, aux_ref):
    gd = gd_ref[...]
    gs = gs_ref[...]
    he = he_ref[:, :64]
    rel = rel_ref[...]
    d2 = jnp.sum(rel * rel, axis=-1, keepdims=True)
    t1 = gd + gs + jnp.dot(he, C_ref[...], preferred_element_type=jnp.float32)
    t1 = t1 + d2 * wd2_ref[...]
    a = jax.nn.silu(t1)
    m = jnp.dot(a, We2_ref[...], preferred_element_type=jnp.float32) + be2_ref[...]
    u = jax.nn.silu(jnp.dot(m, Wc1_ref[...], preferred_element_type=jnp.float32) + bc1_ref[...])
    cw = jnp.tanh(jnp.dot(u, Wc2_ref[...], preferred_element_type=jnp.float32) + bc2_ref[...])
    m_ref[...] = m
    heo = he + jnp.dot(m, Weu_ref[...], preferred_element_type=jnp.float32) + beu_ref[...]
    n = heo.shape[0]
    aux_ref[...] = jnp.concatenate(
        [heo, rel * cw, jnp.ones((n, 1), jnp.float32), jnp.zeros((n, 60), jnp.float32)],
        axis=-1)


def _edge_kernel_body_last(gd_ref, gs_ref, he_ref, rel_ref,
                           C_ref, wd2_ref, We2_ref, be2_ref,
                           Wc1_ref, bc1_ref, Wc2_ref, bc2_ref,
                           Weu_ref, beu_ref, Wed1_ref, bed1_ref, Wed2_ref, bed2_ref,
                           m_ref, aux_ref, pe_ref):
    gd = gd_ref[...]
    gs = gs_ref[...]
    he = he_ref[:, :64]
    rel = rel_ref[...]
    d2 = jnp.sum(rel * rel, axis=-1, keepdims=True)
    t1 = gd + gs + jnp.dot(he, C_ref[...], preferred_element_type=jnp.float32)
    t1 = t1 + d2 * wd2_ref[...]
    a = jax.nn.silu(t1)
    m = jnp.dot(a, We2_ref[...], preferred_element_type=jnp.float32) + be2_ref[...]
    u = jax.nn.silu(jnp.dot(m, Wc1_ref[...], preferred_element_type=jnp.float32) + bc1_ref[...])
    cw = jnp.tanh(jnp.dot(u, Wc2_ref[...], preferred_element_type=jnp.float32) + bc2_ref[...])
    m_ref[...] = m
    n = m.shape[0]
    aux_ref[...] = jnp.concatenate(
        [jnp.zeros((n, 64), jnp.float32), rel * cw,
         jnp.ones((n, 1), jnp.float32), jnp.zeros((n, 60), jnp.float32)],
        axis=-1)
    heo = he + jnp.dot(m, Weu_ref[...], preferred_element_type=jnp.float32) + beu_ref[...]
    pe = jnp.maximum(jnp.dot(heo, Wed1_ref[...], preferred_element_type=jnp.float32) + bed1_ref[...], 0.0)
    pe_ref[...] = jnp.dot(pe, Wed2_ref[...], preferred_element_type=jnp.float32) + bed2_ref[...]


def _edge_layer(gd, gs, he128, rel, C, wd2, We2, be2, Wc1, bc1, Wc2, bc2,
                Weu, beu, last, Wed1=None, bed1=None, Wed2=None, bed2=None):
    E = gd.shape[0]
    ND = gd.shape[1]
    ED = C.shape[0]
    BE = _pick_be(E)
    nb = E // BE
    row = lambda b: (b, 0)
    full = lambda b: (0, 0)
    in_specs = [
        pl.BlockSpec((BE, ND), row),
        pl.BlockSpec((BE, ND), row),
        pl.BlockSpec((BE, ND), row),
        pl.BlockSpec((BE, 3), row),
        pl.BlockSpec((ED, ND), full),
        pl.BlockSpec((1, ND), full),
        pl.BlockSpec((ND, ND), full),
        pl.BlockSpec((1, ND), full),
        pl.BlockSpec((ND, ND), full),
        pl.BlockSpec((1, ND), full),
        pl.BlockSpec((ND, 1), full),
        pl.BlockSpec((1, 1), full),
        pl.BlockSpec((ND, ED), full),
        pl.BlockSpec((1, ED), full),
    ]
    args = [gd, gs, he128, rel, C, wd2, We2, be2, Wc1, bc1, Wc2, bc2, Weu, beu]
    out_shapes = [
        jax.ShapeDtypeStruct((E, ND), jnp.float32),
        jax.ShapeDtypeStruct((E, ND), jnp.float32),
    ]
    out_specs = [
        pl.BlockSpec((BE, ND), row),
        pl.BlockSpec((BE, ND), row),
    ]
    if last:
        ET = Wed2.shape[1]
        in_specs += [
            pl.BlockSpec((ED, ED), full),
            pl.BlockSpec((1, ED), full),
            pl.BlockSpec((ED, ET), full),
            pl.BlockSpec((1, ET), full),
        ]
        args += [Wed1, bed1, Wed2, bed2]
        out_shapes.append(jax.ShapeDtypeStruct((E, ET), jnp.float32))
        out_specs.append(pl.BlockSpec((BE, ET), row))
        body = _edge_kernel_body_last
    else:
        body = _edge_kernel_body
    return pl.pallas_call(
        body,
        grid=(nb,),
        in_specs=in_specs,
        out_specs=out_specs,
        out_shape=out_shapes,
        interpret=INTERPRET,
    )(*args)


def _smear(tt, T, TD):
    offsets = jnp.linspace(0.0, float(T), TD)
    coeff = -0.5 / (offsets[1] - offsets[0]) ** 2
    return jnp.exp(coeff * (tt.astype(jnp.float32)[:, None] - offsets[None, :]) ** 2)


def _ssp(x):
    return jax.nn.softplus(x) - np.log(2.0)


def kernel(protein_pos, protein_v, h_node_pert, pos_pert, h_edge_pert, Wp, bp, Wn, We, We1, be1, We2, be2, Wc1, bc1, Wc2, bc2, Wh1, bh1, Wh2, bh2, Weu, beu, Wv1, bv1, Wv2, bv2, Wnd1, bnd1, Wnd2, bnd2, Wed1, bed1, Wed2, bed2, batch_protein, batch_node, batch_edge, edge_index, t):
    NP = protein_pos.shape[0]
    NL = pos_pert.shape[0]
    E = h_edge_pert.shape[0]
    B = t.shape[0]
    ND = Wp.shape[1]
    TD = ND - Wn.shape[1]
    L = We1.shape[0]
    T = 1000

    src = edge_index[0]
    dst = edge_index[1]
    dstp = jnp.reshape(dst, (E // (_CH * 128), _CH, 128))

    cnt = jnp.clip(jax.ops.segment_sum(jnp.ones((NP, 1)), batch_protein, num_segments=B), 1.0)
    offset = jax.ops.segment_sum(protein_pos, batch_protein, num_segments=B) / cnt
    x = pos_pert - offset[batch_node]
    smear_t = _smear(t, T, TD)
    h = jnp.concatenate([h_node_pert @ Wn, smear_t[batch_node]], axis=-1)
    he128 = jnp.concatenate([h_edge_pert @ We, smear_t[batch_edge],
                             jnp.zeros((E, ND - 64), jnp.float32)], axis=-1)
    hp = protein_v @ Wp + bp
    ctx = jax.ops.segment_sum(hp, batch_protein, num_segments=B) / cnt
    h = h + ctx[batch_node]

    deg = None
    pred_edge = None
    for l in range(L):
        hA = h @ We1[l, :ND] + be1[l]
        hB = h @ We1[l, ND:2 * ND]
        gd = hA[dst]
        gs = hB[src]
        rel = x[dst] - x[src]
        C = We1[l, 2 * ND:-1]
        wd2 = We1[l, -1:]
        last = l == L - 1
        outs = _edge_layer(gd, gs, he128, rel, C, wd2, We2[l], be2[l][None, :],
                           Wc1[l], bc1[l][None, :], Wc2[l], bc2[l][None, :],
                           Weu[l], beu[l][None, :], last,
                           Wed1, bed1[None, :], Wed2, bed2[None, :])
        m, aux = outs[0], outs[1]
        agg, agg_aux = _sc_scatter(m, aux, dstp, NL)
        if deg is None:
            deg = jnp.clip(agg_aux[:, 67:68], 1.0)
        x = x + agg_aux[:, 64:67] / deg
        h = h + jax.nn.silu(jnp.concatenate([h, agg], axis=-1) @ Wh1[l] + bh1[l]) @ Wh2[l] + bh2[l]
        if last:
            pred_edge = outs[2]
        else:
            he128 = aux

    v = _ssp(h @ Wv1 + bv1) @ Wv2 + bv2
    pred_node = jax.nn.relu(v @ Wnd1 + bnd1) @ Wnd2 + bnd2
    pred_pos = x + offset[batch_node]
    return (pred_node, pred_pos, pred_edge)
